# baseline (speedup 1.0000x reference)
import functools

import jax
import jax.numpy as jnp
from jax import lax
from jax.experimental import pallas as pl
from jax.experimental.pallas import tpu as pltpu
from jax.experimental.pallas import tpu_sc as plsc

B = 4096
L = 200
D = 64
LANES = 16
NVR = D // LANES
NFULL = L // LANES
LREM = L - NFULL * LANES

_info = plsc.get_sparse_core_info()
_NC, _NS = _info.num_cores, _info.num_subcores
NW = _NC * _NS
RPW = B // NW


def _tec_body(idx_hbm, table_hbm, out_hbm, idx_all, rows_v, t0_v, out_blk, sem):
    wid = lax.axis_index("s") * _NC + lax.axis_index("c")
    row0 = wid * RPW

    pltpu.sync_copy(idx_hbm.at[pl.ds(row0 * L, RPW * L)],
                    idx_all.at[pl.ds(0, RPW * L)])
    pltpu.sync_copy(table_hbm.at[0], t0_v)

    lane = lax.iota(jnp.int32, LANES)
    last_mask = lane < LREM
    zero = jnp.zeros((LANES,), jnp.float32)
    t0 = [t0_v[pl.ds(k * LANES, LANES)] for k in range(NVR)]

    def row_body(r, carry):
        off = r * L
        c1 = pltpu.async_copy(table_hbm.at[idx_all.at[pl.ds(off, 128)]],
                              rows_v.at[pl.ds(0, 128)], sem)
        c2 = pltpu.async_copy(table_hbm.at[idx_all.at[pl.ds(off + 128, L - 128)]],
                              rows_v.at[pl.ds(128, L - 128)], sem)

        n0v = jnp.zeros((LANES,), jnp.int32)
        for k in range(NFULL):
            v = idx_all[pl.ds(off + k * LANES, LANES)]
            n0v = n0v + plsc.all_reduce_population_count(v == 0)
        v = idx_all[pl.ds(off + NFULL * LANES, LANES)]
        n0v = n0v + plsc.all_reduce_population_count((v == 0) & last_mask)

        c1.wait()
        c2.wait()

        def acc_body(j, accs):
            return tuple(accs[k] + rows_v[j, pl.ds(k * LANES, LANES)]
                         for k in range(NVR))
        accs = lax.fori_loop(0, L, acc_body, (zero, zero, zero, zero), unroll=8)

        n0f = n0v.astype(jnp.float32)
        cntv = jnp.float32(L) - n0f
        scalev = jnp.where(cntv > 0.0, 1.0 / jnp.maximum(cntv, 1.0), 0.0)
        for k in range(NVR):
            out_blk[pl.ds(r * D + k * LANES, LANES)] = (accs[k] - n0f * t0[k]) * scalev
        return carry

    lax.fori_loop(0, RPW, row_body, 0)
    pltpu.sync_copy(out_blk, out_hbm.at[pl.ds(row0 * D, RPW * D)])


_sc_call = functools.partial(
    pl.kernel,
    mesh=plsc.VectorSubcoreMesh(core_axis_name="c", subcore_axis_name="s"),
    out_type=jax.ShapeDtypeStruct((B * D,), jnp.float32),
    compiler_params=pltpu.CompilerParams(
        needs_layout_passes=False, use_tc_tiling_on_sc=False),
    scratch_types=[
        pltpu.VMEM((RPW * L + LANES,), jnp.int32),
        pltpu.VMEM((L, D), jnp.float32),
        pltpu.VMEM((D,), jnp.float32),
        pltpu.VMEM((RPW * D,), jnp.float32),
        pltpu.SemaphoreType.DMA,
    ],
)(_tec_body)


def kernel(inputs, table):
    idx_flat = inputs.reshape(-1).astype(jnp.int32)
    out_flat = _sc_call(idx_flat, table)
    return out_flat.reshape(B, 1, D)


# speedup vs baseline: 1.1950x; 1.1950x over previous
import functools

import jax
import jax.numpy as jnp
from jax import lax
from jax.experimental import pallas as pl
from jax.experimental.pallas import tpu as pltpu
from jax.experimental.pallas import tpu_sc as plsc

B = 4096
L = 200
D = 64
LANES = 16
NVR = D // LANES
NFULL = L // LANES
LREM = L - NFULL * LANES
NBUF = 4

_info = plsc.get_sparse_core_info()
_NC, _NS = _info.num_cores, _info.num_subcores
NW = _NC * _NS
RPW = B // NW


def _tec_body(idx_hbm, table_hbm, out_hbm, idx_all,
              rows0, rows1, rows2, rows3, t0_v, out_blk,
              sem0, sem1, sem2, sem3):
    bufs = (rows0, rows1, rows2, rows3)
    sems = (sem0, sem1, sem2, sem3)
    wid = lax.axis_index("s") * _NC + lax.axis_index("c")
    row0 = wid * RPW

    pltpu.sync_copy(idx_hbm.at[pl.ds(row0 * L, RPW * L)],
                    idx_all.at[pl.ds(0, RPW * L)])
    pltpu.sync_copy(table_hbm.at[0], t0_v)

    lane = lax.iota(jnp.int32, LANES)
    last_mask = lane < LREM
    zero = jnp.zeros((LANES,), jnp.float32)
    t0 = [t0_v[pl.ds(k * LANES, LANES)] for k in range(NVR)]

    def gather_copies(r, buf, sem):
        off = r * L
        c1 = pltpu.make_async_copy(
            table_hbm.at[idx_all.at[pl.ds(off, 128)]],
            buf.at[pl.ds(0, 128)], sem)
        c2 = pltpu.make_async_copy(
            table_hbm.at[idx_all.at[pl.ds(off + 128, L - 128)]],
            buf.at[pl.ds(128, L - 128)], sem)
        return c1, c2

    def row_step(r, buf, sem):
        off = r * L
        n0v = jnp.zeros((LANES,), jnp.int32)
        for k in range(NFULL):
            v = idx_all[pl.ds(off + k * LANES, LANES)]
            n0v = n0v + plsc.all_reduce_population_count(v == 0)
        v = idx_all[pl.ds(off + NFULL * LANES, LANES)]
        n0v = n0v + plsc.all_reduce_population_count((v == 0) & last_mask)

        c1, c2 = gather_copies(r, buf, sem)
        c1.wait()
        c2.wait()

        def acc_body(j, accs):
            return tuple(accs[k] + buf[j, pl.ds(k * LANES, LANES)]
                         for k in range(NVR))
        accs = lax.fori_loop(0, L, acc_body, (zero, zero, zero, zero),
                             unroll=8)

        rn = jnp.minimum(r + NBUF, RPW - 1)
        f1, f2 = gather_copies(rn, buf, sem)
        f1.start()
        f2.start()

        n0f = n0v.astype(jnp.float32)
        cntv = jnp.float32(L) - n0f
        scalev = jnp.where(cntv > 0.0, 1.0 / jnp.maximum(cntv, 1.0), 0.0)
        for k in range(NVR):
            out_blk[pl.ds(r * D + k * LANES, LANES)] = \
                (accs[k] - n0f * t0[k]) * scalev

    for b in range(NBUF):
        c1, c2 = gather_copies(b, bufs[b], sems[b])
        c1.start()
        c2.start()

    def body(i, carry):
        for b in range(NBUF):
            row_step(i * NBUF + b, bufs[b], sems[b])
        return carry

    lax.fori_loop(0, RPW // NBUF, body, 0)

    for b in range(NBUF):
        c1, c2 = gather_copies(0, bufs[b], sems[b])
        c1.wait()
        c2.wait()

    pltpu.sync_copy(out_blk, out_hbm.at[pl.ds(row0 * D, RPW * D)])


_sc_call = functools.partial(
    pl.kernel,
    mesh=plsc.VectorSubcoreMesh(core_axis_name="c", subcore_axis_name="s"),
    out_type=jax.ShapeDtypeStruct((B * D,), jnp.float32),
    compiler_params=pltpu.CompilerParams(
        needs_layout_passes=False, use_tc_tiling_on_sc=False),
    scratch_types=[
        pltpu.VMEM((RPW * L + LANES,), jnp.int32),
        pltpu.VMEM((L, D), jnp.float32),
        pltpu.VMEM((L, D), jnp.float32),
        pltpu.VMEM((L, D), jnp.float32),
        pltpu.VMEM((L, D), jnp.float32),
        pltpu.VMEM((D,), jnp.float32),
        pltpu.VMEM((RPW * D,), jnp.float32),
        pltpu.SemaphoreType.DMA,
        pltpu.SemaphoreType.DMA,
        pltpu.SemaphoreType.DMA,
        pltpu.SemaphoreType.DMA,
    ],
)(_tec_body)


def kernel(inputs, table):
    idx_flat = inputs.reshape(-1).astype(jnp.int32)
    out_flat = _sc_call(idx_flat, table)
    return out_flat.reshape(B, 1, D)
